# baseline (speedup 1.0000x reference)
import functools

import jax
import jax.numpy as jnp
from jax import lax
from jax.experimental import pallas as pl
from jax.experimental.pallas import tpu as pltpu
from jax.experimental.pallas import tpu_sc as plsc

EMBED = 64
G = 128


@functools.partial(jax.jit, static_argnames=("B", "D", "NW", "K"))
def _gather_sc(x_flat, table, *, B, D, NW, K):
    NB = B // G
    nb_per_w = NB // NW
    n_chunks = nb_per_w // K
    mesh = plsc.VectorSubcoreMesh(core_axis_name="c", subcore_axis_name="s")
    info = plsc.get_sparse_core_info()
    nc = info.num_cores

    x2 = x_flat.reshape(NB, G)

    @functools.partial(
        pl.kernel,
        mesh=mesh,
        out_type=jax.ShapeDtypeStruct((NB, G, D), jnp.float32),
        scratch_types=[
            pltpu.VMEM((K, G), jnp.int32),
            pltpu.VMEM((K, G, D), jnp.float32),
            pltpu.SemaphoreType.DMA,
        ],
        compiler_params=pltpu.CompilerParams(use_tc_tiling_on_sc=False),
    )
    def k(x_hbm, table_hbm, out_hbm, idx_v, rows_v, gsem):
        wid = lax.axis_index("s") * nc + lax.axis_index("c")
        row0 = wid * nb_per_w

        def body(i, carry):
            r = row0 + i * K
            pltpu.sync_copy(x_hbm.at[pl.ds(r, K)], idx_v)
            waits = [
                pltpu.async_copy(table_hbm.at[idx_v.at[j]], rows_v.at[j], gsem)
                for j in range(K)
            ]
            for w in waits:
                w.wait()
            pltpu.sync_copy(rows_v, out_hbm.at[pl.ds(r, K)])
            return carry

        lax.fori_loop(0, n_chunks, body, 0)

    return k(x2, table)


def kernel(x, table):
    B = x.shape[0] * x.shape[1]
    out = _gather_sc(x.reshape(B), table, B=B, D=EMBED, NW=32, K=8)
    return out.reshape(x.shape[0], x.shape[1], EMBED)


# speedup vs baseline: 1.3412x; 1.3412x over previous
import functools

import jax
import jax.numpy as jnp
from jax import lax
from jax.experimental import pallas as pl
from jax.experimental.pallas import tpu as pltpu
from jax.experimental.pallas import tpu_sc as plsc

EMBED = 64
G = 128


@functools.partial(jax.jit, static_argnames=("B", "D", "NW", "K"))
def _gather_sc(x2, table, *, B, D, NW, K):
    NB = B // G
    nb_per_w = NB // NW
    n_chunks = nb_per_w // K
    mesh = plsc.VectorSubcoreMesh(core_axis_name="c", subcore_axis_name="s")
    info = plsc.get_sparse_core_info()
    nc = info.num_cores

    @functools.partial(
        pl.kernel,
        mesh=mesh,
        out_type=jax.ShapeDtypeStruct((B, 2 * D), jnp.float32),
        scratch_types=[
            pltpu.VMEM((K, G), jnp.int32),
            pltpu.VMEM((K, G, D), jnp.float32),
            pltpu.SemaphoreType.DMA,
        ],
        compiler_params=pltpu.CompilerParams(use_tc_tiling_on_sc=False),
    )
    def k(x_hbm, table_hbm, out_hbm, idx_v, rows_v, gsem):
        wid = lax.axis_index("s") * nc + lax.axis_index("c")
        row0 = wid * nb_per_w

        def body(i, carry):
            r = row0 + i * K
            pltpu.sync_copy(x_hbm.at[pl.ds(r, K)], idx_v)
            waits = [
                pltpu.async_copy(table_hbm.at[idx_v.at[j]], rows_v.at[j], gsem)
                for j in range(K)
            ]
            for j, w in enumerate(waits):
                w.wait()
                pltpu.sync_copy(
                    rows_v.at[j],
                    out_hbm.at[pl.ds((r + j) * G, G), pl.ds(0, D)],
                )
            return carry

        lax.fori_loop(0, n_chunks, body, 0)

    return k(x2, table)


def kernel(x, table):
    B = x.shape[0] * x.shape[1]
    out = _gather_sc(x.reshape(B // G, G), table, B=B, D=EMBED, NW=32, K=8)
    return out.reshape(x.shape[0], x.shape[1], 2 * EMBED)[:, :, :EMBED]
